# baseline (speedup 1.0000x reference)
import jax
import jax.numpy as jnp
from jax.experimental import pallas as pl

N = 1024
B = 32
D = 64
PRE_K = 4
BB = 2
IN_SZ = 1 + 2 * D


def _bf(x):
    return x.astype(jnp.bfloat16)


def _cell_kernel(in_ref, hx_ref, gw_ref, gb_ref, w_ref, bias_ref, r_ref,
                 aw_ref, ab_ref, pw_ref, pb_ref, s_ref, y_ref, hxo_ref):
    h = hx_ref[...]
    r = r_ref[...]
    aw = aw_ref[...]

    hr = h + r[None]
    hrb = _bf(hr).astype(jnp.float32)
    awb = _bf(aw).astype(jnp.float32)
    logits = jnp.sum(hrb * awb[None, None], axis=(2, 3)) + ab_ref[0]
    m = jnp.max(logits, axis=1, keepdims=True)
    e = jnp.exp(logits - m)
    wts = e / jnp.sum(e, axis=1, keepdims=True)

    x0 = jnp.concatenate(
        [jnp.concatenate([in_ref[i], h[i, 3], h[i, 2]], axis=1)
         for i in range(BB)], axis=1)
    s_bf = s_ref[...]
    x1 = jnp.dot(s_bf, _bf(x0), preferred_element_type=jnp.float32)
    x2 = 2.0 * jnp.dot(s_bf, _bf(x1),
                       preferred_element_type=jnp.float32) - x0

    xc = jnp.concatenate(
        [jnp.concatenate([x0[:, i * IN_SZ:(i + 1) * IN_SZ],
                          x1[:, i * IN_SZ:(i + 1) * IN_SZ],
                          x2[:, i * IN_SZ:(i + 1) * IN_SZ]], axis=1)
         for i in range(BB)], axis=1)
    gc = jnp.dot(_bf(xc), gw_ref[...],
                 preferred_element_type=jnp.float32)
    gb4 = jnp.concatenate([gb_ref[...]] * BB)
    gc = gc + gb4[None, :]

    conv = jnp.where(gc >= 0, gc, 0.01 * gc)
    out = jnp.dot(_bf(conv), w_ref[...],
                  preferred_element_type=jnp.float32)

    hxo_ref[:, 0:3] = h[:, 1:4]
    pwb = _bf(pw_ref[...]).astype(jnp.float32)
    bias = bias_ref[...]
    for i in range(BB):
        att_i = wts[i, 0] * hr[i, 0]
        att_i = att_i + wts[i, 1] * hr[i, 1]
        att_i = att_i + wts[i, 2] * hr[i, 2]
        att_i = att_i + wts[i, 3] * hr[i, 3]
        out_i = out[:, i * D:(i + 1) * D] + bias + att_i
        hxo_ref[i, 3] = out_i
        outb = _bf(out_i).astype(jnp.float32)
        y_ref[i] = jnp.sum(outb * pwb[None, :], axis=1,
                           keepdims=True) + pb_ref[0]


def kernel(inputs, hx_k, gconv_w, gconv_b, W, b, R, att_w, att_b, proj_w,
           proj_b, support):
    gw_r = gconv_w.reshape(IN_SZ, 3, D).transpose(1, 0, 2).reshape(3 * IN_SZ, D)
    gw_bd = _bf(jnp.kron(jnp.eye(BB, dtype=gconv_w.dtype), gw_r))
    w_bd = _bf(jnp.kron(jnp.eye(BB, dtype=W.dtype), W))
    s_bf = _bf(support)
    awm = att_w.reshape(N, D)
    pw = proj_w.reshape(D)

    y, hx_new = pl.pallas_call(
        _cell_kernel,
        grid=(B // BB,),
        in_specs=[
            pl.BlockSpec((BB, N, 1), lambda i: (i, 0, 0)),
            pl.BlockSpec((BB, PRE_K, N, D), lambda i: (i, 0, 0, 0)),
            pl.BlockSpec((3 * IN_SZ * BB, D * BB), lambda i: (0, 0)),
            pl.BlockSpec((D,), lambda i: (0,)),
            pl.BlockSpec((D * BB, D * BB), lambda i: (0, 0)),
            pl.BlockSpec((N, D), lambda i: (0, 0)),
            pl.BlockSpec((PRE_K, N, D), lambda i: (0, 0, 0)),
            pl.BlockSpec((N, D), lambda i: (0, 0)),
            pl.BlockSpec((1,), lambda i: (0,)),
            pl.BlockSpec((D,), lambda i: (0,)),
            pl.BlockSpec((1,), lambda i: (0,)),
            pl.BlockSpec((N, N), lambda i: (0, 0)),
        ],
        out_specs=[
            pl.BlockSpec((BB, N, 1), lambda i: (i, 0, 0)),
            pl.BlockSpec((BB, PRE_K, N, D), lambda i: (i, 0, 0, 0)),
        ],
        out_shape=[
            jax.ShapeDtypeStruct((B, N, 1), jnp.float32),
            jax.ShapeDtypeStruct((B, PRE_K, N, D), jnp.float32),
        ],
    )(inputs[:, :, None], hx_k[0], gw_bd, gconv_b, w_bd, b, R, awm, att_b,
      pw, proj_b, s_bf)
    return y.reshape(B, N), hx_new[None]
